# baseline (speedup 1.0000x reference)
import functools

import jax
import jax.numpy as jnp
from jax import lax
from jax.experimental import pallas as pl
from jax.experimental.pallas import tpu as pltpu
from jax.experimental.pallas import tpu_sc as plsc

V, D, B, L = 1000000, 64, 16384, 50

NC, NS = 2, 16
NW = NC * NS
NB = B // NW
NE = NB * L
CB = 4
ROWS = CB * L
NCH = NB // CB
NPAIR = NCH // 2
GROUPS = NB // 16


def _sc_pool_body(xf_hbm, emb_hbm, w_hbm, fcb_hbm, out_hbm,
                  idx_v, buf0, buf1, out_v, w_v, fcb_v, sem0, sem1):
    wid = lax.axis_index("s") * NC + lax.axis_index("c")
    base = wid * NB

    pltpu.sync_copy(xf_hbm.at[pl.ds(wid * NE, NE)], idx_v)
    pltpu.sync_copy(w_hbm, w_v)
    pltpu.sync_copy(fcb_hbm, fcb_v)

    w4 = [w_v[pl.ds(16 * k, 16)] for k in range(4)]
    fcb16 = fcb_v[...]
    lane = lax.iota(jnp.int32, 16)
    lane15 = lane == 15

    def _fire(c, buf, sem):
        pltpu.async_copy(emb_hbm.at[idx_v.at[pl.ds(c * ROWS, ROWS)]], buf, sem)

    def _drain(buf, sem):
        pltpu.make_async_copy(
            emb_hbm.at[idx_v.at[pl.ds(0, ROWS)]], buf, sem).wait()

    def _process(c, buf):
        def batch_body(j, carry):
            b = c * CB + j
            acc = [jnp.zeros((16,), jnp.float32) for _ in range(4)]
            for l in range(L):
                r = j * L + l
                for k in range(4):
                    acc[k] = acc[k] + buf[r, pl.ds(16 * k, 16)] * w4[k]
            s = (acc[0] + acc[1]) + (acc[2] + acc[3])
            cum = plsc.cumsum(s)
            plsc.store_scatter(out_v, [jnp.full((16,), b, jnp.int32)],
                               cum, mask=lane15)
            return carry
        lax.fori_loop(0, CB, batch_body, 0)

    _fire(0, buf0, sem0)

    def pair_body(cc, carry):
        c0 = cc * 2
        _fire(c0 + 1, buf1, sem1)
        _drain(buf0, sem0)
        _process(c0, buf0)

        @pl.when(cc < NPAIR - 1)
        def _():
            _fire(c0 + 2, buf0, sem0)

        _drain(buf1, sem1)
        _process(c0 + 1, buf1)
        return carry

    lax.fori_loop(0, NPAIR, pair_body, 0)

    def fin_body(g, carry):
        bvec = g * (16 * L) + lane * L
        cnt = jnp.zeros((16,), jnp.float32)
        one = jnp.ones((16,), jnp.float32)
        zero = jnp.zeros((16,), jnp.float32)
        for l in range(L):
            tok = plsc.load_gather(idx_v, [bvec + l])
            cnt = cnt + jnp.where(tok != 0, one, zero)
        raw = out_v[pl.ds(g * 16, 16)]
        out_v[pl.ds(g * 16, 16)] = raw / jnp.maximum(cnt, one) + fcb16
        return carry

    lax.fori_loop(0, GROUPS, fin_body, 0)
    pltpu.sync_copy(out_v, out_hbm.at[pl.ds(base, NB)])


@functools.lru_cache(maxsize=1)
def _make_sc_pool():
    mesh = plsc.VectorSubcoreMesh(
        core_axis_name="c", subcore_axis_name="s", num_cores=NC)
    return pl.kernel(
        _sc_pool_body,
        out_type=jax.ShapeDtypeStruct((B,), jnp.float32),
        mesh=mesh,
        scratch_types=[
            pltpu.VMEM((NE,), jnp.int32),
            pltpu.VMEM((ROWS, D), jnp.float32),
            pltpu.VMEM((ROWS, D), jnp.float32),
            pltpu.VMEM((NB,), jnp.float32),
            pltpu.VMEM((D,), jnp.float32),
            pltpu.VMEM((16,), jnp.float32),
            pltpu.SemaphoreType.DMA,
            pltpu.SemaphoreType.DMA,
        ],
        compiler_params=pltpu.CompilerParams(
            needs_layout_passes=False, use_tc_tiling_on_sc=False),
    )


def kernel(x, emb, fc_w, fc_b):
    xf = x.reshape(B * L)
    w64 = fc_w.reshape(D)
    fcb16 = jnp.broadcast_to(fc_b.astype(jnp.float32), (16,))
    return _make_sc_pool()(xf, emb, w64, fcb16)
